# baseline (speedup 1.0000x reference)
import functools

import jax
import jax.numpy as jnp
from jax import lax
from jax.experimental import pallas as pl
from jax.experimental.pallas import tpu as pltpu
from jax.experimental.pallas import tpu_sc as plsc

N = 4096
K = 16
EPS = 0.1
NEG_INF = float("-inf")


def _topk_body(s_ref, idx_ref, cnt_ref):
    v = s_ref[...]
    c = v.shape[1]
    rows = jax.lax.broadcasted_iota(jnp.int32, (N, c), 0)
    cnt_ref[...] = jnp.sum((v > EPS).astype(jnp.float32), axis=0,
                           keepdims=True)
    am = jnp.full((1, c), -1, jnp.int32)
    for k in range(K):
        v = jnp.where(rows == am, NEG_INF, v)
        am = jnp.argmax(v, axis=0).astype(jnp.int32).reshape(1, c)
        idx_ref[k : k + 1, :] = am


def _topk(S):
    C = 128
    grid = (N // C,)
    return pl.pallas_call(
        _topk_body,
        grid=grid,
        in_specs=[pl.BlockSpec((N, C), lambda j: (0, j))],
        out_specs=[
            pl.BlockSpec((K, C), lambda j: (0, j)),
            pl.BlockSpec((1, C), lambda j: (0, j)),
        ],
        out_shape=[
            jax.ShapeDtypeStruct((K, N), jnp.int32),
            jax.ShapeDtypeStruct((1, N), jnp.float32),
        ],
        compiler_params=pltpu.CompilerParams(
            dimension_semantics=("arbitrary",)
        ),
    )(S)


def _build_body(cnt_ref, idx_ref, h_ref, degd_ref, binv_ref):
    rb = pl.program_id(0)
    r = h_ref.shape[0]
    ks = jax.lax.broadcasted_iota(jnp.int32, (K, 1), 0).astype(jnp.float32)
    mv = ks < cnt_ref[...]
    mvb = mv.astype(jnp.bfloat16)
    iv = idx_ref[...].astype(jnp.int16)
    rows = (jax.lax.broadcasted_iota(jnp.int32, (r, 1), 0) + rb * r
            ).astype(jnp.int16)
    zero = jnp.zeros((r, N), jnp.bfloat16)
    acc = zero
    for k in range(K):
        acc = acc + jnp.where(iv[k : k + 1, :] == rows, mvb[k : k + 1, :],
                              zero)
    h_ref[...] = acc
    degd_ref[...] = jnp.sum(acc.astype(jnp.float32), axis=1, keepdims=True)

    @pl.when(rb == 0)
    def _():
        degb = jnp.sum(mv.astype(jnp.float32), axis=0, keepdims=True)
        binv_ref[...] = jnp.where(degb > 0, 1.0 / jnp.maximum(degb, 1e-9), 0.0)


def _build(cnt, idx):
    R = 512
    grid = (N // R,)
    return pl.pallas_call(
        _build_body,
        grid=grid,
        in_specs=[
            pl.BlockSpec((1, N), lambda i: (0, 0)),
            pl.BlockSpec((K, N), lambda i: (0, 0)),
        ],
        out_specs=[
            pl.BlockSpec((R, N), lambda i: (i, 0)),
            pl.BlockSpec((R, 1), lambda i: (i, 0)),
            pl.BlockSpec((1, N), lambda i: (0, 0)),
        ],
        out_shape=[
            jax.ShapeDtypeStruct((N, N), jnp.bfloat16),
            jax.ShapeDtypeStruct((N, 1), jnp.float32),
            jax.ShapeDtypeStruct((1, N), jnp.float32),
        ],
        compiler_params=pltpu.CompilerParams(
            dimension_semantics=("arbitrary",)
        ),
    )(cnt, idx)


def _agg_t_body(h_ref, x_ref, binvc_ref, out_ref, *, nk):
    kb = pl.program_id(1)
    prod = jax.lax.dot_general(
        h_ref[...].astype(jnp.float32), x_ref[...], (((0,), (0,)), ((), ())),
        preferred_element_type=jnp.float32,
    )

    @pl.when(kb == 0)
    def _():
        out_ref[...] = prod

    @pl.when(kb != 0)
    def _():
        out_ref[...] += prod

    @pl.when(kb == nk - 1)
    def _():
        out_ref[...] *= binvc_ref[...]


def _agg_t(H, x_bf, binvc):
    F = x_bf.shape[1]
    J = 1024
    R = 1024
    nk = N // R
    grid = (N // J, nk)
    return pl.pallas_call(
        functools.partial(_agg_t_body, nk=nk),
        grid=grid,
        in_specs=[
            pl.BlockSpec((R, J), lambda j, k: (k, j)),
            pl.BlockSpec((R, F), lambda j, k: (k, 0)),
            pl.BlockSpec((J, 1), lambda j, k: (j, 0)),
        ],
        out_specs=pl.BlockSpec((J, F), lambda j, k: (j, 0)),
        out_shape=jax.ShapeDtypeStruct((N, F), jnp.float32),
        compiler_params=pltpu.CompilerParams(
            dimension_semantics=("parallel", "arbitrary")
        ),
    )(H, x_bf, binvc)


def _scatter_body(h_ref, he_ref, degd_ref, bias_ref, out_ref, *,
                  nk, final_tanh):
    kb = pl.program_id(1)
    prod = jnp.dot(h_ref[...].astype(jnp.float32), he_ref[...],
                   preferred_element_type=jnp.float32)

    @pl.when(kb == 0)
    def _():
        out_ref[...] = prod

    @pl.when(kb != 0)
    def _():
        out_ref[...] += prod

    @pl.when(kb == nk - 1)
    def _():
        dv = degd_ref[...]
        dinv = jnp.where(dv > 0, 1.0 / jnp.maximum(dv, 1e-9), 0.0)
        r = out_ref[...] * dinv + bias_ref[...]
        out_ref[...] = jnp.tanh(r) if final_tanh else r


def _scatter(H, he, degd, bias, final_tanh):
    F = he.shape[1]
    R = 1024
    J = 1024
    nk = N // J
    grid = (N // R, nk)
    return pl.pallas_call(
        functools.partial(_scatter_body, nk=nk, final_tanh=final_tanh),
        grid=grid,
        in_specs=[
            pl.BlockSpec((R, J), lambda i, k: (i, k)),
            pl.BlockSpec((J, F), lambda i, k: (k, 0)),
            pl.BlockSpec((R, 1), lambda i, k: (i, 0)),
            pl.BlockSpec((1, F), lambda i, k: (0, 0)),
        ],
        out_specs=pl.BlockSpec((R, F), lambda i, k: (i, 0)),
        out_shape=jax.ShapeDtypeStruct((N, F), jnp.float32),
        compiler_params=pltpu.CompilerParams(
            dimension_semantics=("parallel", "arbitrary")
        ),
    )(H, he, degd, bias)


def _mlp_body(z_ref, w1_ref, b1_ref, w2_ref, out_ref):
    mid = jnp.dot(z_ref[...], w1_ref[...], preferred_element_type=jnp.float32)
    mid = jnp.maximum(mid + b1_ref[...], 0.0)
    out_ref[...] = jnp.dot(mid, w2_ref[...], preferred_element_type=jnp.float32)


def _mlp(z, W1, b1, W2):
    IN_F, HID = W1.shape
    CODE = W2.shape[1]
    R = 512
    grid = (N // R,)
    return pl.pallas_call(
        _mlp_body,
        grid=grid,
        in_specs=[
            pl.BlockSpec((R, IN_F), lambda i: (i, 0)),
            pl.BlockSpec((IN_F, HID), lambda i: (0, 0)),
            pl.BlockSpec((1, HID), lambda i: (0, 0)),
            pl.BlockSpec((HID, CODE), lambda i: (0, 0)),
        ],
        out_specs=pl.BlockSpec((R, CODE), lambda i: (i, 0)),
        out_shape=jax.ShapeDtypeStruct((N, CODE), jnp.float32),
        compiler_params=pltpu.CompilerParams(
            dimension_semantics=("arbitrary",)
        ),
    )(z, W1, b1, W2)


def _prep_body(cnt_ref, idx_ref, safe_ref, binv_ref, binvc_ref):
    ks = jax.lax.broadcasted_iota(jnp.int32, (K, 1), 0).astype(jnp.float32)
    mv = ks < cnt_ref[...]
    safe = jnp.where(mv, idx_ref[...], N)
    safe_ref[...] = jnp.transpose(safe, (1, 0))
    degb = jnp.sum(mv.astype(jnp.float32), axis=0, keepdims=True)
    binv = jnp.where(degb > 0, 1.0 / jnp.maximum(degb, 1e-9), 0.0)
    binv_ref[...] = binv
    binvc_ref[...] = jnp.transpose(binv, (1, 0))


def _prep(cnt, idx):
    C = 512
    grid = (N // C,)
    return pl.pallas_call(
        _prep_body,
        grid=grid,
        in_specs=[
            pl.BlockSpec((1, C), lambda j: (0, j)),
            pl.BlockSpec((K, C), lambda j: (0, j)),
        ],
        out_specs=[
            pl.BlockSpec((C, K), lambda j: (j, 0)),
            pl.BlockSpec((1, C), lambda j: (0, j)),
            pl.BlockSpec((C, 1), lambda j: (j, 0)),
        ],
        out_shape=[
            jax.ShapeDtypeStruct((N, K), jnp.int32),
            jax.ShapeDtypeStruct((1, N), jnp.float32),
            jax.ShapeDtypeStruct((N, 1), jnp.float32),
        ],
        compiler_params=pltpu.CompilerParams(
            dimension_semantics=("arbitrary",)
        ),
    )(cnt, idx)


_NW = 32
_JPW = N // _NW


_JPG = 8
_NG = _JPW // _JPG


def _sc_conv(t_pad, safe_flat, safe_grp, binv, zeros, F):
    mesh = plsc.VectorSubcoreMesh(core_axis_name="c", subcore_axis_name="s")

    @functools.partial(
        pl.kernel,
        out_type=[
            jax.ShapeDtypeStruct((N, F), jnp.float32),
            jax.ShapeDtypeStruct((N, F), jnp.float32),
        ],
        mesh=mesh,
        scratch_types=[
            pltpu.VMEM((_JPW * K,), jnp.int32),
            pltpu.VMEM((_NG, _JPG * K), jnp.int32),
            pltpu.VMEM((1, _JPW + 16), jnp.float32),
            pltpu.VMEM((_JPG * K, F), jnp.float32),
            pltpu.VMEM((_JPG * K, F), jnp.float32),
            pltpu.VMEM((_JPG * K, F), jnp.float32),
            pltpu.VMEM((N // 16, F), jnp.float32),
            pltpu.VMEM_SHARED((N + 1, F), jnp.float32),
            pltpu.SemaphoreType.DMA,
            pltpu.SemaphoreType.DMA,
        ],
    )
    def body(t_hbm, safe1_hbm, safe2_hbm, binv_hbm, zeros_hbm, out_a, out_b,
             safe1_v, safe2_v, binv_v, rows_a, rows_b, sbuf_v, stage_v,
             acc_sh, sem_a, sem_b):
        cid = lax.axis_index("c")
        sid = lax.axis_index("s")
        wid = cid * 16 + sid
        j0 = wid * _JPW

        @pl.when(sid == 0)
        def _():
            pltpu.sync_copy(zeros_hbm, acc_sh)

        pltpu.sync_copy(safe1_hbm.at[pl.ds(j0 * K, _JPW * K)], safe1_v)
        pltpu.sync_copy(safe2_hbm.at[pl.ds(wid * _NG, _NG)], safe2_v)
        pltpu.sync_copy(binv_hbm.at[:, pl.ds(j0, _JPW)],
                        binv_v.at[:, pl.ds(0, _JPW)])
        plsc.subcore_barrier()

        GK = _JPG * K

        def gstart(g, buf, sem):
            pltpu.make_async_copy(
                t_hbm.at[safe1_v.at[pl.ds(g * GK, GK)]], buf, sem).start()

        def gwait(buf, sem):
            pltpu.make_async_copy(t_hbm.at[pl.ds(0, GK)], buf, sem).wait()

        def process(g, buf):
            for q in range(_JPG):
                bv = binv_v[0, pl.ds(g * _JPG + q, 16)][0]
                for c in range(F // 16):
                    sl = pl.ds(c * 16, 16)
                    terms = [buf[q * K + kk, sl] for kk in range(K)]
                    while len(terms) > 1:
                        terms = [terms[i] + terms[i + 1]
                                 for i in range(0, len(terms), 2)]
                    acc = terms[0] * bv
                    for kk in range(K):
                        sbuf_v[q * K + kk, sl] = acc
            pltpu.sync_copy(sbuf_v, acc_sh.at[safe2_v.at[g]], add=True)

        gstart(0, rows_a, sem_a)

        def pair(i, carry):
            g0 = i * 2
            gstart(g0 + 1, rows_b, sem_b)
            gwait(rows_a, sem_a)
            process(g0, rows_a)

            @pl.when(i < _NG // 2 - 1)
            def _():
                gstart(g0 + 2, rows_a, sem_a)

            gwait(rows_b, sem_b)
            process(g0 + 1, rows_b)
            return carry

        lax.fori_loop(0, _NG // 2, pair, 0)
        plsc.subcore_barrier()
        r0 = sid * (N // 16)
        pltpu.sync_copy(acc_sh.at[pl.ds(r0, N // 16)], stage_v)

        @pl.when(cid == 0)
        def _():
            pltpu.sync_copy(stage_v, out_a.at[pl.ds(r0, N // 16)])

        @pl.when(cid == 1)
        def _():
            pltpu.sync_copy(stage_v, out_b.at[pl.ds(r0, N // 16)])

    return body(t_pad, safe_flat, safe_grp, binv, zeros)


def _finish_body(a_ref, b_ref, degd_ref, bias_ref, out_ref):
    F = out_ref.shape[1]
    dv = degd_ref[...]
    dinv = jnp.where(dv > 0, 1.0 / jnp.maximum(dv, 1e-9), 0.0)
    s = a_ref[:, :F] + b_ref[:, :F]
    out_ref[...] = jnp.tanh(s * dinv + bias_ref[...])


def _finish(za, zb, degd, bias):
    Fp = za.shape[1]
    F = bias.shape[1]
    R = 1024
    grid = (N // R,)
    return pl.pallas_call(
        _finish_body,
        grid=grid,
        in_specs=[
            pl.BlockSpec((R, Fp), lambda i: (i, 0)),
            pl.BlockSpec((R, Fp), lambda i: (i, 0)),
            pl.BlockSpec((R, 1), lambda i: (i, 0)),
            pl.BlockSpec((1, F), lambda i: (0, 0)),
        ],
        out_specs=pl.BlockSpec((R, F), lambda i: (i, 0)),
        out_shape=jax.ShapeDtypeStruct((N, F), jnp.float32),
        compiler_params=pltpu.CompilerParams(
            dimension_semantics=("arbitrary",)
        ),
    )(za, zb, degd, bias)


def kernel(x, S, W1, b1, W2, b2):
    idx, cnt = _topk(S)
    H, degd, binv = _build(cnt, idx)
    safeT, binv_p, binvc = _prep(cnt, idx)
    zero_b = jnp.zeros((1, x.shape[1]), jnp.float32)
    heb = _agg_t(H, x, binvc)
    z = _scatter(H, heb, degd, zero_b, final_tanh=False)
    t = _mlp(z, W1, b1.reshape(1, -1), W2)
    t_pad = jnp.pad(t, ((0, 1), (0, 128 - t.shape[1])))
    zeros = jnp.zeros((N + 1, 128), jnp.float32)
    safe_flat = safeT.reshape(-1)
    safe_grp = safeT.reshape(N * K // (_JPG * K), _JPG * K)
    z2a, z2b = _sc_conv(t_pad, safe_flat, safe_grp, binv_p, zeros, 128)
    code = _finish(z2a, z2b, degd, b2.reshape(1, -1))
    return code


# speedup vs baseline: 1.1809x; 1.1809x over previous
import functools

import jax
import jax.numpy as jnp
from jax import lax
from jax.experimental import pallas as pl
from jax.experimental.pallas import tpu as pltpu
from jax.experimental.pallas import tpu_sc as plsc

N = 4096
K = 16
EPS = 0.1
NEG_INF = float("-inf")


def _topk_body(s_ref, idx_ref, cnt_ref):
    v = s_ref[...]
    c = v.shape[1]
    rows = jax.lax.broadcasted_iota(jnp.int32, (N, c), 0)
    cnt_ref[...] = jnp.sum((v > EPS).astype(jnp.float32), axis=0,
                           keepdims=True)
    am = jnp.full((1, c), -1, jnp.int32)
    for k in range(K):
        v = jnp.where(rows == am, NEG_INF, v)
        am = jnp.argmax(v, axis=0).astype(jnp.int32).reshape(1, c)
        idx_ref[k : k + 1, :] = am


def _topk(S):
    C = 256
    grid = (N // C,)
    return pl.pallas_call(
        _topk_body,
        grid=grid,
        in_specs=[pl.BlockSpec((N, C), lambda j: (0, j))],
        out_specs=[
            pl.BlockSpec((K, C), lambda j: (0, j)),
            pl.BlockSpec((1, C), lambda j: (0, j)),
        ],
        out_shape=[
            jax.ShapeDtypeStruct((K, N), jnp.int32),
            jax.ShapeDtypeStruct((1, N), jnp.float32),
        ],
        compiler_params=pltpu.CompilerParams(
            dimension_semantics=("arbitrary",)
        ),
    )(S)


def _build_body(cnt_ref, idx_ref, h_ref, degd_ref, binv_ref):
    rb = pl.program_id(0)
    r = h_ref.shape[0]
    ks = jax.lax.broadcasted_iota(jnp.int32, (K, 1), 0).astype(jnp.float32)
    mv = ks < cnt_ref[...]
    mvb = mv.astype(jnp.bfloat16)
    iv = idx_ref[...].astype(jnp.int16)
    rows = (jax.lax.broadcasted_iota(jnp.int32, (r, 1), 0) + rb * r
            ).astype(jnp.int16)
    acc = jnp.zeros((r, N), jnp.bfloat16)
    for k in range(K):
        acc = jnp.where(iv[k : k + 1, :] == rows, mvb[k : k + 1, :], acc)
    h_ref[...] = acc
    degd_ref[...] = jnp.sum(acc.astype(jnp.float32), axis=1, keepdims=True)

    @pl.when(rb == 0)
    def _():
        degb = jnp.sum(mv.astype(jnp.float32), axis=0, keepdims=True)
        binv_ref[...] = jnp.where(degb > 0, 1.0 / jnp.maximum(degb, 1e-9), 0.0)


def _build(cnt, idx):
    R = 512
    grid = (N // R,)
    return pl.pallas_call(
        _build_body,
        grid=grid,
        in_specs=[
            pl.BlockSpec((1, N), lambda i: (0, 0)),
            pl.BlockSpec((K, N), lambda i: (0, 0)),
        ],
        out_specs=[
            pl.BlockSpec((R, N), lambda i: (i, 0)),
            pl.BlockSpec((R, 1), lambda i: (i, 0)),
            pl.BlockSpec((1, N), lambda i: (0, 0)),
        ],
        out_shape=[
            jax.ShapeDtypeStruct((N, N), jnp.bfloat16),
            jax.ShapeDtypeStruct((N, 1), jnp.float32),
            jax.ShapeDtypeStruct((1, N), jnp.float32),
        ],
        compiler_params=pltpu.CompilerParams(
            dimension_semantics=("arbitrary",)
        ),
    )(cnt, idx)


def _agg_t_body(h_ref, x_ref, binvc_ref, out_ref, *, nk):
    kb = pl.program_id(1)
    prod = jax.lax.dot_general(
        h_ref[...].astype(jnp.float32), x_ref[...], (((0,), (0,)), ((), ())),
        preferred_element_type=jnp.float32,
    )

    @pl.when(kb == 0)
    def _():
        out_ref[...] = prod

    @pl.when(kb != 0)
    def _():
        out_ref[...] += prod

    @pl.when(kb == nk - 1)
    def _():
        out_ref[...] *= binvc_ref[...]


def _agg_t(H, x_bf, binvc):
    F = x_bf.shape[1]
    J = 1024
    R = 1024
    nk = N // R
    grid = (N // J, nk)
    return pl.pallas_call(
        functools.partial(_agg_t_body, nk=nk),
        grid=grid,
        in_specs=[
            pl.BlockSpec((R, J), lambda j, k: (k, j)),
            pl.BlockSpec((R, F), lambda j, k: (k, 0)),
            pl.BlockSpec((J, 1), lambda j, k: (j, 0)),
        ],
        out_specs=pl.BlockSpec((J, F), lambda j, k: (j, 0)),
        out_shape=jax.ShapeDtypeStruct((N, F), jnp.float32),
        compiler_params=pltpu.CompilerParams(
            dimension_semantics=("parallel", "arbitrary")
        ),
    )(H, x_bf, binvc)


def _scatter_body(h_ref, he_ref, degd_ref, bias_ref, out_ref, *,
                  nk, final_tanh):
    kb = pl.program_id(1)
    prod = jnp.dot(h_ref[...].astype(jnp.float32), he_ref[...],
                   preferred_element_type=jnp.float32)

    @pl.when(kb == 0)
    def _():
        out_ref[...] = prod

    @pl.when(kb != 0)
    def _():
        out_ref[...] += prod

    @pl.when(kb == nk - 1)
    def _():
        dv = degd_ref[...]
        dinv = jnp.where(dv > 0, 1.0 / jnp.maximum(dv, 1e-9), 0.0)
        r = out_ref[...] * dinv + bias_ref[...]
        out_ref[...] = jnp.tanh(r) if final_tanh else r


def _scatter(H, he, degd, bias, final_tanh):
    F = he.shape[1]
    R = 1024
    J = 1024
    nk = N // J
    grid = (N // R, nk)
    return pl.pallas_call(
        functools.partial(_scatter_body, nk=nk, final_tanh=final_tanh),
        grid=grid,
        in_specs=[
            pl.BlockSpec((R, J), lambda i, k: (i, k)),
            pl.BlockSpec((J, F), lambda i, k: (k, 0)),
            pl.BlockSpec((R, 1), lambda i, k: (i, 0)),
            pl.BlockSpec((1, F), lambda i, k: (0, 0)),
        ],
        out_specs=pl.BlockSpec((R, F), lambda i, k: (i, 0)),
        out_shape=jax.ShapeDtypeStruct((N, F), jnp.float32),
        compiler_params=pltpu.CompilerParams(
            dimension_semantics=("parallel", "arbitrary")
        ),
    )(H, he, degd, bias)


def _mlp_body(z_ref, w1_ref, b1_ref, w2_ref, out_ref):
    mid = jnp.dot(z_ref[...], w1_ref[...], preferred_element_type=jnp.float32)
    mid = jnp.maximum(mid + b1_ref[...], 0.0)
    out_ref[...] = jnp.dot(mid, w2_ref[...], preferred_element_type=jnp.float32)


def _mlp(z, W1, b1, W2):
    IN_F, HID = W1.shape
    CODE = W2.shape[1]
    R = 512
    grid = (N // R,)
    return pl.pallas_call(
        _mlp_body,
        grid=grid,
        in_specs=[
            pl.BlockSpec((R, IN_F), lambda i: (i, 0)),
            pl.BlockSpec((IN_F, HID), lambda i: (0, 0)),
            pl.BlockSpec((1, HID), lambda i: (0, 0)),
            pl.BlockSpec((HID, CODE), lambda i: (0, 0)),
        ],
        out_specs=pl.BlockSpec((R, CODE), lambda i: (i, 0)),
        out_shape=jax.ShapeDtypeStruct((N, CODE), jnp.float32),
        compiler_params=pltpu.CompilerParams(
            dimension_semantics=("arbitrary",)
        ),
    )(z, W1, b1, W2)


def _prep_body(cnt_ref, idx_ref, safe_ref, binv_ref, binvc_ref):
    ks = jax.lax.broadcasted_iota(jnp.int32, (K, 1), 0).astype(jnp.float32)
    mv = ks < cnt_ref[...]
    safe = jnp.where(mv, idx_ref[...], N)
    safe_ref[...] = jnp.transpose(safe, (1, 0))
    degb = jnp.sum(mv.astype(jnp.float32), axis=0, keepdims=True)
    binv = jnp.where(degb > 0, 1.0 / jnp.maximum(degb, 1e-9), 0.0)
    binv_ref[...] = binv
    binvc_ref[...] = jnp.transpose(binv, (1, 0))


def _prep(cnt, idx):
    C = 512
    grid = (N // C,)
    return pl.pallas_call(
        _prep_body,
        grid=grid,
        in_specs=[
            pl.BlockSpec((1, C), lambda j: (0, j)),
            pl.BlockSpec((K, C), lambda j: (0, j)),
        ],
        out_specs=[
            pl.BlockSpec((C, K), lambda j: (j, 0)),
            pl.BlockSpec((1, C), lambda j: (0, j)),
            pl.BlockSpec((C, 1), lambda j: (j, 0)),
        ],
        out_shape=[
            jax.ShapeDtypeStruct((N, K), jnp.int32),
            jax.ShapeDtypeStruct((1, N), jnp.float32),
            jax.ShapeDtypeStruct((N, 1), jnp.float32),
        ],
        compiler_params=pltpu.CompilerParams(
            dimension_semantics=("arbitrary",)
        ),
    )(cnt, idx)


_NW = 32
_JPW = N // _NW


_JPG = 8
_NG = _JPW // _JPG


def _sc_conv(t_pad, safe_flat, safe_grp, binv, zeros, F):
    mesh = plsc.VectorSubcoreMesh(core_axis_name="c", subcore_axis_name="s")

    @functools.partial(
        pl.kernel,
        out_type=[
            jax.ShapeDtypeStruct((N, F), jnp.float32),
            jax.ShapeDtypeStruct((N, F), jnp.float32),
        ],
        mesh=mesh,
        scratch_types=[
            pltpu.VMEM((_JPW * K,), jnp.int32),
            pltpu.VMEM((_NG, _JPG * K), jnp.int32),
            pltpu.VMEM((1, _JPW + 16), jnp.float32),
            pltpu.VMEM((_JPG * K, F), jnp.float32),
            pltpu.VMEM((_JPG * K, F), jnp.float32),
            pltpu.VMEM((_JPG * K, F), jnp.float32),
            pltpu.VMEM((N // 16, F), jnp.float32),
            pltpu.VMEM_SHARED((N + 1, F), jnp.float32),
            pltpu.SemaphoreType.DMA,
            pltpu.SemaphoreType.DMA,
        ],
    )
    def body(t_hbm, safe1_hbm, safe2_hbm, binv_hbm, zeros_hbm, out_a, out_b,
             safe1_v, safe2_v, binv_v, rows_a, rows_b, sbuf_v, stage_v,
             acc_sh, sem_a, sem_b):
        cid = lax.axis_index("c")
        sid = lax.axis_index("s")
        wid = cid * 16 + sid
        j0 = wid * _JPW

        @pl.when(sid == 0)
        def _():
            pltpu.sync_copy(zeros_hbm, acc_sh)

        pltpu.sync_copy(safe1_hbm.at[pl.ds(j0 * K, _JPW * K)], safe1_v)
        pltpu.sync_copy(safe2_hbm.at[pl.ds(wid * _NG, _NG)], safe2_v)
        pltpu.sync_copy(binv_hbm.at[:, pl.ds(j0, _JPW)],
                        binv_v.at[:, pl.ds(0, _JPW)])
        plsc.subcore_barrier()

        GK = _JPG * K

        def gstart(g, buf, sem):
            pltpu.make_async_copy(
                t_hbm.at[safe1_v.at[pl.ds(g * GK, GK)]], buf, sem).start()

        def gwait(buf, sem):
            pltpu.make_async_copy(t_hbm.at[pl.ds(0, GK)], buf, sem).wait()

        def process(g, buf):
            for q in range(_JPG):
                bv = binv_v[0, pl.ds(g * _JPG + q, 16)][0]
                for c in range(F // 16):
                    sl = pl.ds(c * 16, 16)
                    terms = [buf[q * K + kk, sl] for kk in range(K)]
                    while len(terms) > 1:
                        terms = [terms[i] + terms[i + 1]
                                 for i in range(0, len(terms), 2)]
                    acc = terms[0] * bv
                    for kk in range(K):
                        sbuf_v[q * K + kk, sl] = acc
            pltpu.sync_copy(sbuf_v, acc_sh.at[safe2_v.at[g]], add=True)

        gstart(0, rows_a, sem_a)

        def pair(i, carry):
            g0 = i * 2
            gstart(g0 + 1, rows_b, sem_b)
            gwait(rows_a, sem_a)
            process(g0, rows_a)

            @pl.when(i < _NG // 2 - 1)
            def _():
                gstart(g0 + 2, rows_a, sem_a)

            gwait(rows_b, sem_b)
            process(g0 + 1, rows_b)
            return carry

        lax.fori_loop(0, _NG // 2, pair, 0)
        plsc.subcore_barrier()
        r0 = sid * (N // 16)
        pltpu.sync_copy(acc_sh.at[pl.ds(r0, N // 16)], stage_v)

        @pl.when(cid == 0)
        def _():
            pltpu.sync_copy(stage_v, out_a.at[pl.ds(r0, N // 16)])

        @pl.when(cid == 1)
        def _():
            pltpu.sync_copy(stage_v, out_b.at[pl.ds(r0, N // 16)])

    return body(t_pad, safe_flat, safe_grp, binv, zeros)


def _finish_body(a_ref, b_ref, degd_ref, bias_ref, out_ref):
    F = out_ref.shape[1]
    dv = degd_ref[...]
    dinv = jnp.where(dv > 0, 1.0 / jnp.maximum(dv, 1e-9), 0.0)
    s = a_ref[:, :F] + b_ref[:, :F]
    out_ref[...] = jnp.tanh(s * dinv + bias_ref[...])


def _finish(za, zb, degd, bias):
    Fp = za.shape[1]
    F = bias.shape[1]
    R = 1024
    grid = (N // R,)
    return pl.pallas_call(
        _finish_body,
        grid=grid,
        in_specs=[
            pl.BlockSpec((R, Fp), lambda i: (i, 0)),
            pl.BlockSpec((R, Fp), lambda i: (i, 0)),
            pl.BlockSpec((R, 1), lambda i: (i, 0)),
            pl.BlockSpec((1, F), lambda i: (0, 0)),
        ],
        out_specs=pl.BlockSpec((R, F), lambda i: (i, 0)),
        out_shape=jax.ShapeDtypeStruct((N, F), jnp.float32),
        compiler_params=pltpu.CompilerParams(
            dimension_semantics=("arbitrary",)
        ),
    )(za, zb, degd, bias)


def kernel(x, S, W1, b1, W2, b2):
    idx, cnt = _topk(S)
    H, degd, binv = _build(cnt, idx)
    safeT, binv_p, binvc = _prep(cnt, idx)
    zero_b = jnp.zeros((1, x.shape[1]), jnp.float32)
    heb = _agg_t(H, x, binvc)
    z = _scatter(H, heb, degd, zero_b, final_tanh=False)
    t = _mlp(z, W1, b1.reshape(1, -1), W2)
    t_pad = jnp.pad(t, ((0, 1), (0, 128 - t.shape[1])))
    zeros = jnp.zeros((N + 1, 128), jnp.float32)
    safe_flat = safeT.reshape(-1)
    safe_grp = safeT.reshape(N * K // (_JPG * K), _JPG * K)
    z2a, z2b = _sc_conv(t_pad, safe_flat, safe_grp, binv_p, zeros, 128)
    code = _finish(z2a, z2b, degd, b2.reshape(1, -1))
    return code
